# baseline (speedup 1.0000x reference)
import functools

import jax
import jax.numpy as jnp
from jax import lax
from jax.experimental import pallas as pl
from jax.experimental.pallas import tpu as pltpu
from jax.experimental.pallas import tpu_sc as plsc

BATCH = 16384
EMBED_DIM = 32
NUM_CORES = 2
NUM_SUBCORES = 16
NUM_WORKERS = NUM_CORES * NUM_SUBCORES
BPW = BATCH // NUM_WORKERS
IDX_CHUNK = 128
NCHUNK = BPW // IDX_CHUNK
NGROUP = BPW // 16


@functools.partial(
    pl.kernel,
    mesh=plsc.VectorSubcoreMesh(core_axis_name="c", subcore_axis_name="s"),
    compiler_params=pltpu.CompilerParams(
        needs_layout_passes=False, use_tc_tiling_on_sc=False),
    out_type=jax.ShapeDtypeStruct((BATCH,), jnp.float32),
    scratch_types=[
        pltpu.VMEM((NCHUNK, IDX_CHUNK), jnp.int32),
        pltpu.VMEM((NCHUNK, IDX_CHUNK), jnp.int32),
        pltpu.VMEM((BPW, EMBED_DIM), jnp.float32),
        pltpu.VMEM((BPW, EMBED_DIM), jnp.float32),
        pltpu.VMEM((BPW,), jnp.float32),
        pltpu.SemaphoreType.DMA,
        pltpu.SemaphoreType.DMA,
    ],
)
def _mf_score_sc(uidx_hbm, iidx_hbm, utab_hbm, itab_hbm, out_hbm,
                 uidx_v, iidx_v, urows_v, irows_v, out_v, usem, isem):
    wid = lax.axis_index("s") * NUM_CORES + lax.axis_index("c")
    base = wid * BPW

    pltpu.sync_copy(uidx_hbm.at[pl.ds(wid * NCHUNK, NCHUNK)], uidx_v)
    pltpu.sync_copy(iidx_hbm.at[pl.ds(wid * NCHUNK, NCHUNK)], iidx_v)

    copies = []
    for j in range(NCHUNK):
        copies.append(pltpu.async_copy(
            utab_hbm.at[uidx_v.at[j]],
            urows_v.at[pl.ds(j * IDX_CHUNK, IDX_CHUNK)], usem))
        copies.append(pltpu.async_copy(
            itab_hbm.at[iidx_v.at[j]],
            irows_v.at[pl.ds(j * IDX_CHUNK, IDX_CHUNK)], isem))
    for c in copies:
        c.wait()

    def group_body(g, carry):
        rows = g * 16 + lax.iota(jnp.int32, 16)
        acc = jnp.zeros((16,), jnp.float32)
        for d in range(EMBED_DIM):
            cols = jnp.full((16,), d, jnp.int32)
            u = plsc.load_gather(urows_v, [rows, cols])
            v = plsc.load_gather(irows_v, [rows, cols])
            acc = acc + u * v
        out_v[pl.ds(g * 16, 16)] = acc
        return carry

    lax.fori_loop(0, NGROUP, group_body, 0)

    pltpu.sync_copy(out_v, out_hbm.at[pl.ds(base, BPW)])


def kernel(user_idx, item_idx, user_table, item_table):
    uidx = user_idx.astype(jnp.int32).reshape(NUM_WORKERS * NCHUNK, IDX_CHUNK)
    iidx = item_idx.astype(jnp.int32).reshape(NUM_WORKERS * NCHUNK, IDX_CHUNK)
    return _mf_score_sc(uidx, iidx, user_table, item_table)


# speedup vs baseline: 12.6082x; 12.6082x over previous
import functools

import jax
import jax.numpy as jnp
from jax import lax
from jax.experimental import pallas as pl
from jax.experimental.pallas import tpu as pltpu
from jax.experimental.pallas import tpu_sc as plsc

NUM_ROWS = 1000000
BATCH = 16384
EMBED_DIM = 32
SUBLANE = 8
DTILE = EMBED_DIM // SUBLANE
NUM_CORES = 2
NUM_SUBCORES = 16
NUM_WORKERS = NUM_CORES * NUM_SUBCORES
BPW = BATCH // NUM_WORKERS
LPP = 16
NPASS = BPW // LPP
SLAB = LPP * 16


@functools.partial(
    pl.kernel,
    mesh=plsc.VectorSubcoreMesh(core_axis_name="c", subcore_axis_name="s"),
    compiler_params=pltpu.CompilerParams(needs_layout_passes=False),
    out_type=jax.ShapeDtypeStruct((BATCH,), jnp.float32),
    scratch_types=[
        pltpu.VMEM((BPW + LPP,), jnp.int32),
        pltpu.VMEM((BPW + LPP,), jnp.int32),
        pltpu.VMEM((DTILE, SUBLANE, SLAB), jnp.float32),
        pltpu.VMEM((DTILE, SUBLANE, SLAB), jnp.float32),
        pltpu.VMEM((DTILE, SUBLANE, SLAB), jnp.float32),
        pltpu.VMEM((DTILE, SUBLANE, SLAB), jnp.float32),
        pltpu.VMEM((BPW,), jnp.float32),
        pltpu.SemaphoreType.DMA,
        pltpu.SemaphoreType.DMA,
        pltpu.SemaphoreType.DMA,
        pltpu.SemaphoreType.DMA,
    ],
)
def _mf_score_sc(uidx_hbm, iidx_hbm, utab_hbm, itab_hbm, out_hbm,
                 uidx_v, iidx_v, uvalA, ivalA, uvalB, ivalB, out_v,
                 usemA, isemA, usemB, isemB):
    wid = lax.axis_index("s") * NUM_CORES + lax.axis_index("c")
    base = wid * BPW

    pltpu.sync_copy(uidx_hbm.at[pl.ds(base, BPW)], uidx_v.at[pl.ds(0, BPW)])
    pltpu.sync_copy(iidx_hbm.at[pl.ds(base, BPW)], iidx_v.at[pl.ds(0, BPW)])
    uidx_v[pl.ds(BPW, LPP)] = jnp.zeros((LPP,), jnp.int32)
    iidx_v[pl.ds(BPW, LPP)] = jnp.zeros((LPP,), jnp.int32)

    def fire(p, uslab, islab, usem, isem):
        def fk(k, carry):
            j = p * LPP + k
            uv = uidx_v[pl.ds(j, 16)]
            iv = iidx_v[pl.ds(j, 16)]
            ub = pl.multiple_of((uv[0] >> 4) << 4, 16)
            ib = pl.multiple_of((iv[0] >> 4) << 4, 16)
            pltpu.async_copy(utab_hbm.at[:, :, pl.ds(ub, 16)],
                             uslab.at[:, :, pl.ds(k * 16, 16)], usem)
            pltpu.async_copy(itab_hbm.at[:, :, pl.ds(ib, 16)],
                             islab.at[:, :, pl.ds(k * 16, 16)], isem)
            return carry
        lax.fori_loop(0, LPP, fk, 0)

    def drain(uslab, islab, usem, isem):
        pltpu.make_async_copy(utab_hbm.at[:, :, pl.ds(0, SLAB)],
                              uslab, usem).wait()
        pltpu.make_async_copy(itab_hbm.at[:, :, pl.ds(0, SLAB)],
                              islab, isem).wait()

    def compute(p, uslab, islab):
        u16 = uidx_v[pl.ds(p * LPP, 16)]
        i16 = iidx_v[pl.ds(p * LPP, 16)]
        lane_base = lax.iota(jnp.int32, 16) * 16
        ulanes = lane_base + (u16 & 15)
        ilanes = lane_base + (i16 & 15)
        acc = jnp.zeros((16,), jnp.float32)
        for t in range(DTILE):
            tt = jnp.full((16,), t, jnp.int32)
            for s in range(SUBLANE):
                ss = jnp.full((16,), s, jnp.int32)
                u = plsc.load_gather(uslab, [tt, ss, ulanes])
                v = plsc.load_gather(islab, [tt, ss, ilanes])
                acc = acc + u * v
        out_v[pl.ds(p * LPP, 16)] = acc

    fire(0, uvalA, ivalA, usemA, isemA)

    def body(h, carry):
        p = h * 2
        fire(p + 1, uvalB, ivalB, usemB, isemB)
        drain(uvalA, ivalA, usemA, isemA)
        compute(p, uvalA, ivalA)
        fire(p + 2, uvalA, ivalA, usemA, isemA)
        drain(uvalB, ivalB, usemB, isemB)
        compute(p + 1, uvalB, ivalB)
        return carry

    lax.fori_loop(0, NPASS // 2 - 1, body, 0)

    fire(NPASS - 1, uvalB, ivalB, usemB, isemB)
    drain(uvalA, ivalA, usemA, isemA)
    compute(NPASS - 2, uvalA, ivalA)
    drain(uvalB, ivalB, usemB, isemB)
    compute(NPASS - 1, uvalB, ivalB)

    pltpu.sync_copy(out_v, out_hbm.at[pl.ds(base, BPW)])


def kernel(user_idx, item_idx, user_table, item_table):
    ut3 = user_table.T.reshape(DTILE, SUBLANE, NUM_ROWS)
    it3 = item_table.T.reshape(DTILE, SUBLANE, NUM_ROWS)
    return _mf_score_sc(user_idx.astype(jnp.int32),
                        item_idx.astype(jnp.int32), ut3, it3)
